# baseline (speedup 1.0000x reference)
import functools

import jax
import jax.numpy as jnp
import numpy as np
from jax import lax
from jax.experimental import pallas as pl
from jax.experimental.pallas import tpu as pltpu
from jax.experimental.pallas import tpu_sc as plsc

N = 2048; K = 16; L = 9; H = 32; H2 = 64; NH = 8; NL = 4; AC = 91; EC = 32
E = N * K
NB = 128
EB = NB * K
GRID = N // NB
F32 = jnp.float32


def _fp(shape):
    return jax.ShapeDtypeStruct(shape, F32)


def _bs(shape, edge=False):
    nd = len(shape)
    return pl.BlockSpec(shape, lambda i: (i,) + (0,) * (nd - 1))


def _ws(shape):
    nd = len(shape)
    return pl.BlockSpec(shape, lambda i: (0,) * nd)


def _brd16(x):
    nb = x.shape[0]
    return jnp.broadcast_to(x[:, None, :], (nb, K, x.shape[1])).reshape(nb * K, x.shape[1])


def _sc_gather(table, idx2):
    T, D = table.shape
    nrow, gc = idx2.shape
    Etot = nrow * gc
    NW = 32
    nch = nrow // NW
    mesh = plsc.VectorSubcoreMesh(core_axis_name="c", subcore_axis_name="s")

    @functools.partial(
        pl.kernel, mesh=mesh,
        out_type=jax.ShapeDtypeStruct((Etot, D), F32),
        scratch_types=[
            pltpu.VMEM((nch, gc), jnp.int32),
            pltpu.VMEM((gc, D), F32),
            pltpu.VMEM((gc, D), F32),
            pltpu.SemaphoreType.DMA,
            pltpu.SemaphoreType.DMA,
        ],
    )
    def k(table_hbm, idx_hbm, out_hbm, idx_v, buf0, buf1, gsem, wsem):
        wid = lax.axis_index("s") * 2 + lax.axis_index("c")
        base = wid * nch
        pltpu.sync_copy(idx_hbm.at[pl.ds(base, nch)], idx_v)
        bufs = (buf0, buf1)
        g = [None, None]
        w = [None, None]
        for c in range(nch):
            b = c % 2
            if w[b] is not None:
                w[b].wait()
            g[b] = pltpu.async_copy(table_hbm.at[idx_v.at[c]], bufs[b], gsem)
            ob = 1 - b
            if c >= 1:
                g[ob].wait()
                w[ob] = pltpu.async_copy(
                    bufs[ob], out_hbm.at[pl.ds((base + c - 1) * gc, gc)], wsem)
        last = (nch - 1) % 2
        g[last].wait()
        w[last] = pltpu.async_copy(
            bufs[last], out_hbm.at[pl.ds((base + nch - 1) * gc, gc)], wsem)
        if w[1 - last] is not None:
            w[1 - last].wait()
        w[last].wait()

    return k(table, idx2)


def _pack_bf16(vlo, vhi):
    ulo = lax.bitcast_convert_type(vlo, jnp.uint32) >> 16
    uhi = lax.bitcast_convert_type(vhi, jnp.uint32) & jnp.uint32(0xFFFF0000)
    return lax.bitcast_convert_type(uhi | ulo, F32)


def _unpack_bf16(w):
    u = lax.bitcast_convert_type(w, jnp.uint32)
    vlo = lax.bitcast_convert_type(u << 16, F32)
    vhi = lax.bitcast_convert_type(u & jnp.uint32(0xFFFF0000), F32)
    return vlo, vhi


def _to_rows(x, w):
    return jnp.concatenate([x[:, w * l:w * (l + 1)] for l in range(L)], axis=0)


def _from_rows(q, w):
    nb = q.shape[0] // L
    return jnp.concatenate([q[nb * l:nb * (l + 1), :] for l in range(L)], axis=1)


def _geom_body(xT, bbT, mk, bbf_o, xm4_o, xmT_o):
    eps = 1e-8
    xr = xT[...]
    X = bbT[...]
    m = mk[...]
    laneI = lax.broadcasted_iota(jnp.int32, (1, N), 1)
    zero1 = jnp.zeros((1, 1), F32)

    def row(A, i):
        return A[i:i + 1, :]

    def shiftL(r):
        return jnp.concatenate([r[:, 1:], zero1], axis=1)

    def shiftR(r):
        return jnp.concatenate([zero1, r[:, :-1]], axis=1)

    xmrows = [jnp.where(m > 0, 1e9, row(xr, c)) for c in range(3)]
    xmT_o[...] = jnp.concatenate(xmrows + [jnp.zeros((5, N), F32)], axis=0)
    xm4_o[...] = jnp.transpose(
        jnp.concatenate(xmrows + [jnp.zeros((1, N), F32)], axis=0), (1, 0))

    dx = [shiftL(row(xr, c)) - row(xr, c) for c in range(3)]
    nrmf = jnp.sqrt(dx[0] * dx[0] + dx[1] * dx[1] + dx[2] * dx[2])
    fu = [d / (nrmf + eps) for d in dx]
    fwd = [jnp.where(laneI <= N - 2, f, 0.0) for f in fu]
    bwd = [-shiftR(f) for f in fwd]

    bv = [row(X, 3 + c) - row(X, 0 + c) for c in range(3)]
    cv = [row(X, 6 + c) - row(X, 3 + c) for c in range(3)]
    av = [bv[1] * cv[2] - bv[2] * cv[1],
          bv[2] * cv[0] - bv[0] * cv[2],
          bv[0] * cv[1] - bv[1] * cv[0]]
    vcb = [(-0.58273431 * av[c] + 0.56802827 * bv[c] - 0.54067466 * cv[c]
            + row(X, 3 + c)) - row(xr, c) for c in range(3)]

    def sh1(A):
        return jnp.concatenate([A[1:2], A[2:3], shiftL(A[0:1])], axis=0)

    Xf = [jnp.concatenate([row(X, c), row(X, 3 + c), row(X, 6 + c)], axis=0)
          for c in range(3)]
    dXf = [sh1(Xf[c]) - Xf[c] for c in range(3)]
    nrm = jnp.sqrt(dXf[0] * dXf[0] + dXf[1] * dXf[1] + dXf[2] * dXf[2])
    U = [d / (nrm + eps) for d in dXf]
    u1 = [sh1(U[c]) for c in range(3)]
    u0 = [sh1(u1[c]) for c in range(3)]

    def crossn(a, b):
        c0 = a[1] * b[2] - a[2] * b[1]
        c1 = a[2] * b[0] - a[0] * b[2]
        c2 = a[0] * b[1] - a[1] * b[0]
        nn = jnp.sqrt(c0 * c0 + c1 * c1 + c2 * c2)
        return [c0 / (nn + eps), c1 / (nn + eps), c2 / (nn + eps)]

    n2 = crossn(U, u1)
    n1 = crossn(u1, u0)
    cosD = jnp.clip(n2[0] * n1[0] + n2[1] * n1[1] + n2[2] * n1[2],
                    -1 + 1e-7, 1 - 1e-7)
    sgn = jnp.sign(U[0] * n1[0] + U[1] * n1[1] + U[2] * n1[2])
    sinD = sgn * jnp.sqrt(1.0 - cosD * cosD)
    valid = laneI <= N - 2
    cosV = jnp.where(valid, cosD, 1.0)
    sinV = jnp.where(valid, sinD, 0.0)
    cp0 = jnp.where(laneI == 0, 1.0, shiftR(cosV[2:3]))
    sp0 = shiftR(sinV[2:3])
    cosP = [cp0, cosV[0:1], cosV[1:2]]
    sinP = [sp0, sinV[0:1], sinV[1:2]]

    rows = cosP + sinP + [jnp.zeros((1, N), F32)]
    for c in range(3):
        l1 = [row(X, a * 3 + c) - row(xr, c) for a in range(4)]
        l1 += [fwd[c], bwd[c], vcb[c]]
        rows += [jnp.nan_to_num(r) for r in l1]
    rows += [jnp.zeros((4, N), F32)]
    bbf_o[...] = jnp.transpose(jnp.concatenate(rows, axis=0), (1, 0))


RK = 256


def _knn_body(xm4, xmT, bbf, W1, nbr_o, dist_o, rel_o, bbp_o):
    i = pl.program_id(0)
    xb = xm4[...]
    xT = xmT[...]
    d2 = ((xb[:, 0:1] - xT[0:1, :]) ** 2 + (xb[:, 1:2] - xT[1:2, :]) ** 2
          + (xb[:, 2:3] - xT[2:3, :]) ** 2)
    rowg = i * RK + lax.broadcasted_iota(jnp.int32, (RK, 1), 0)
    colI = lax.broadcasted_iota(jnp.int32, (RK, N), 1)
    d2 = d2 + jnp.where(colI == rowg, 1e12, 0.0)
    idxs = []
    ds = []
    for _ in range(K):
        m = jnp.min(d2, axis=1, keepdims=True)
        j = jnp.min(jnp.where(d2 == m, colI, N), axis=1, keepdims=True)
        idxs.append(j)
        ds.append(m)
        d2 = jnp.where(colI == j, jnp.float32(jnp.inf), d2)
    nbr = jnp.concatenate(idxs, axis=1)
    nbr_o[...] = nbr
    dist_o[...] = jnp.sqrt(jnp.concatenate(ds, axis=1))
    rel_o[...] = (nbr - rowg).astype(F32)
    bbp_o[...] = jnp.dot(bbf[...], W1[...], preferred_element_type=F32)


def _embed_body(g0, dist, rel, latf, W2, beb, Lv0, quad0,
                ef_o, res_o, t1_o, sdst_o):
    g0v = g0[...][:, :L * H]
    d = dist[...]
    r = rel[...]
    mu = lax.broadcasted_iota(jnp.int32, (1, 16), 1).astype(F32) * (20.0 / 15.0)
    sig = 20.0 / 16.0
    rbf = jnp.exp(-(((d - mu) / sig) ** 2))
    freq = jnp.exp(lax.broadcasted_iota(jnp.int32, (1, 8), 1).astype(F32) * 2.0 * (-np.log(10000.0) / 16.0))
    ang = r * freq
    kk = jnp.round(ang * 0.15915494309189535)
    t = (ang - kk * 6.2831855) - kk * (-1.7484556e-7)
    t2 = t * t
    sn = t * (1 + t2 * (-1.0 / 6 + t2 * (1.0 / 120 + t2 * (-1.0 / 5040
         + t2 * (1.0 / 362880 + t2 * (-1.0 / 39916800 + t2 * (1.0 / 6227020800)))))))
    cs = 1 + t2 * (-0.5 + t2 * (1.0 / 24 + t2 * (-1.0 / 720 + t2 * (1.0 / 40320
         + t2 * (-1.0 / 3628800 + t2 * (1.0 / 479001600 + t2 * (-1.0 / 87178291200)))))))
    ef = jnp.concatenate([rbf, cs, sn], axis=1)
    ef_o[...] = ef
    m = jax.nn.relu(g0v + jnp.dot(ef, W2[...], preferred_element_type=F32) + beb[...])
    bbe = jnp.mean(m.reshape(NB, K, L * H), axis=1)
    la = latf[...]
    pieces = []
    for l in range(L):
        pieces.append(bbe[:, l * H:(l + 1) * H])
        pieces.append(la[:, l * H:(l + 1) * H])
    res = jnp.concatenate(pieces, axis=1)
    res_o[...] = res
    v0 = _from_rows(jnp.dot(_to_rows(res, H2), Lv0[...], preferred_element_type=F32), H2)
    quad = jnp.dot(res[:, :H2], quad0[...], preferred_element_type=F32)
    vp = _pack_bf16(v0[:, :288], v0[:, 288:])
    t1_o[...] = jnp.concatenate(
        [quad[:, :32], jnp.zeros((NB, 32), F32), vp, jnp.zeros((NB, 32), F32)], axis=1)
    sdst_o[...] = quad[:, 32:64]


def _layer_body(first, last, g, ef_p, p_p, sdst, res, A1e, La2, Lo, Lf1, Lf2,
                LvN, quadW, E1e, P576,
                res_o, t1_o, sdst_o, ef_o, p_o):
    gb = g[...]
    vlo, vhi = _unpack_bf16(gb[:, 64:352])
    v576 = jnp.concatenate([vlo, vhi], axis=1)
    if first:
        ef = ef_p[...]
    else:
        ef = ef_p[...] + jnp.tanh(gb[:, 32:64] + p_p[...])
    ef_o[...] = ef
    a1 = jax.nn.relu(gb[:, 0:32] + _brd16(sdst[...])
                     + jnp.dot(ef, A1e[...], preferred_element_type=F32))
    a = jnp.dot(a1, La2[...], preferred_element_type=F32)
    a3 = a.reshape(NB, K, NH)
    amax = jnp.max(a3, axis=1, keepdims=True)
    ex = jnp.exp(a3 - amax)
    attn = (ex / jnp.sum(ex, axis=1, keepdims=True)).reshape(NB * K, NH)
    att576 = jnp.dot(attn, P576[...], preferred_element_type=F32)
    msg = jnp.sum((v576 * att576).reshape(NB, K, L * H2), axis=1)
    msgr = _to_rows(msg, H2)
    if last:
        resp = _from_rows(jnp.dot(msgr, Lo[...], preferred_element_type=F32), H)
        res_o[...] = jnp.dot(resp, LvN[...], preferred_element_type=F32)
        return
    resm = res[...] + _from_rows(jnp.dot(msgr, Lo[...], preferred_element_type=F32), H2)
    resmr = _to_rows(resm, H2)
    ffh = jax.nn.gelu(jnp.dot(resmr, Lf1[...], preferred_element_type=F32))
    resn = resm + _from_rows(jnp.dot(ffh, Lf2[...], preferred_element_type=F32), H2)
    res_o[...] = resn
    vn = _from_rows(jnp.dot(_to_rows(resn, H2), LvN[...], preferred_element_type=F32), H2)
    quad = jnp.dot(resn[:, :H2], quadW[...], preferred_element_type=F32)
    vp = _pack_bf16(vn[:, :288], vn[:, 288:])
    t1_o[...] = jnp.concatenate(
        [quad[:, 0:32], quad[:, 32:64], vp, jnp.zeros((NB, 32), F32)], axis=1)
    sdst_o[...] = quad[:, 64:96]
    p_o[...] = _brd16(quad[:, 96:128]) + jnp.dot(ef, E1e[...], preferred_element_type=F32)


def _out_body(g5, ef4, Wo2, boa, lng, lnb, Ws1, bs1, Ws2, bs2, atom_o, logits_o):
    m2 = jax.nn.relu(g5[...] + jnp.dot(ef4[...], Wo2[...], preferred_element_type=F32) + boa[...])
    atom = jnp.mean(m2.reshape(NB, K, 384), axis=1)
    atom_o[...] = atom
    invf = atom[:, :AC]
    mu = jnp.mean(invf, axis=-1, keepdims=True)
    xc = invf - mu
    var = jnp.mean(xc * xc, axis=-1, keepdims=True)
    h = xc / jnp.sqrt(var + 1e-5) * lng[...] + lnb[...]
    h = jax.nn.relu(jnp.dot(h, Ws1[...], preferred_element_type=F32) + bs1[...])
    lg = jnp.dot(h, Ws2[...], preferred_element_type=F32) + bs2[...]
    mx = jnp.max(lg, axis=-1, keepdims=True)
    s = lg - mx
    logits_o[...] = s - jnp.log(jnp.sum(jnp.exp(s), axis=-1, keepdims=True))


def _tc_call(body, in_arrs, in_specs, out_shapes, out_specs):
    return pl.pallas_call(
        body,
        grid=(GRID,),
        in_specs=in_specs,
        out_specs=out_specs,
        out_shape=out_shapes,
    )(*in_arrs)


def kernel(x, bb, x_mask, latent, W_eb, b_eb, La1, La2, Lv, Lo, Lf1, Lf2, Le,
           Pa1, Pa2, Pv, Po, W_oa, b_oa, ln_g, ln_b, Ws1, bs1, Ws2, bs2):
    xT8 = jnp.pad(x.T, ((0, 5), (0, 0)))
    bbT16 = jnp.pad(bb.transpose(1, 2, 0).reshape(12, N), ((0, 4), (0, 0)))
    mk = x_mask.astype(F32).reshape(1, N)
    bbf, xm4, xmT = pl.pallas_call(
        _geom_body,
        out_shape=(_fp((N, 32)), _fp((N, 4)), _fp((8, N))),
    )(xT8, bbT16, mk)


    W1 = jnp.pad(W_eb[:28], ((0, 4), (0, 96)))
    W2 = W_eb[28:]
    beb = b_eb.reshape(1, L * H)
    A1s = La1[:, :H2]; A1d = La1[:, H2:2 * H2]; A1e = La1[:, 2 * H2:]
    E1s = Le[:, :H2]; E1d = Le[:, H2:2 * H2]; E1e = Le[:, 2 * H2:]
    P1s = Pa1[:H2]; P1d = Pa1[H2:2 * H2]; P1e = Pa1[2 * H2:]
    Wo1 = jnp.pad(W_oa[:L * H], ((0, 0), (0, 20)))
    Wo2 = jnp.pad(W_oa[L * H:], ((0, 0), (0, 20)))
    boa = jnp.pad(b_oa, (0, 20)).reshape(1, 384)
    fcol = np.arange(L * H2)
    P576 = jnp.asarray((((fcol % H2) // 8)[None, :] == np.arange(NH)[:, None]).astype(np.float32))
    quad0 = jnp.concatenate([A1s[0], A1d[0]], axis=1)
    latf = latent.reshape(N, L * H)

    nbr, dist, rel, bbp = pl.pallas_call(
        _knn_body,
        grid=(N // RK,),
        in_specs=[_bs((RK, 4)), _ws((8, N)), _bs((RK, 32)), _ws((32, 384))],
        out_specs=[_bs((RK, K)), _bs((RK, K)), _bs((RK, K)), _bs((RK, 384))],
        out_shape=(jax.ShapeDtypeStruct((N, K), jnp.int32),
                   _fp((N, K)), _fp((N, K)), _fp((N, 384))),
    )(xm4, xmT, bbf, W1)
    idx2 = nbr.reshape(E // 128, 128)
    dist_e = dist.reshape(E, 1)
    rel_e = rel.reshape(E, 1)
    g0 = _gather(bbp, idx2)

    ef0, res0, t1, sdst = _tc_call(
        _embed_body,
        [g0, dist_e, rel_e, latf, W2, beb, Lv[0], quad0],
        [_bs((EB, 384)), _bs((EB, 1)), _bs((EB, 1)), _bs((NB, 288)),
         _ws((32, 288)), _ws((1, 288)), _ws((64, 64)), _ws((64, 64))],
        (_fp((E, 32)), _fp((N, 576)), _fp((N, 384)), _fp((N, 32))),
        [_bs((EB, 32)), _bs((NB, 576)), _bs((NB, 384)), _bs((NB, 32))],
    )

    res, ef, p = res0, ef0, None
    for i in range(NL + 1):
        first = i == 0
        last = i == NL
        g = _gather(t1, idx2)
        if last:
            lvn = Wo1
            quadw = jnp.zeros((64, 128), F32)
            e1e = jnp.zeros((32, 32), F32)
            lo_i, la2_i, a1e_i = Po, Pa2, P1e
            lf1_i = jnp.zeros((64, 32), F32); lf2_i = jnp.zeros((32, 64), F32)
        else:
            a1e_i, la2_i, lo_i, lf1_i, lf2_i = A1e[i], La2[i], Lo[i], Lf1[i], Lf2[i]
            if i == NL - 1:
                lvn = Pv
                quadw = jnp.concatenate([P1s, E1s[i], P1d, E1d[i]], axis=1)
            else:
                lvn = Lv[i + 1]
                quadw = jnp.concatenate([A1s[i + 1], E1s[i], A1d[i + 1], E1d[i]], axis=1)
            e1e = E1e[i]
        p_in = ef if first else p
        body = functools.partial(_layer_body, first, last)
        outs = _tc_call(
            body,
            [g, ef, p_in, sdst, res, a1e_i, la2_i, lo_i, lf1_i, lf2_i,
             lvn, quadw, e1e, P576],
            [_bs((EB, 384)), _bs((EB, 32)), _bs((EB, 32)), _bs((NB, 32)),
             _bs((NB, 576)), _ws((32, 32)), _ws((32, NH)), _ws(lo_i.shape),
             _ws((64, 32)), _ws((32, 64)), _ws(lvn.shape), _ws((64, 128)),
             _ws((32, 32)), _ws((NH, 576))],
            (_fp((N, 384)) if last else _fp((N, 576)),
             _fp((N, 384)), _fp((N, 32)), _fp((E, 32)), _fp((E, 32))),
            [_bs((NB, 384)) if last else _bs((NB, 576)),
             _bs((NB, 384)), _bs((NB, 32)), _bs((EB, 32)), _bs((EB, 32))],
        )
        if last:
            oat, _, _, ef4, _ = outs
        else:
            res, t1, sdst, ef, p = outs

    g5 = _gather(oat, idx2)
    atom, logits = _tc_call(
        _out_body,
        [g5, ef4, Wo2, boa, ln_g.reshape(1, AC), ln_b.reshape(1, AC),
         Ws1, bs1.reshape(1, AC), Ws2, bs2.reshape(1, 20)],
        [_bs((EB, 384)), _bs((EB, 32)), _ws((32, 384)), _ws((1, 384)),
         _ws((1, AC)), _ws((1, AC)), _ws((AC, AC)), _ws((1, AC)),
         _ws((AC, 20)), _ws((1, 20))],
        (_fp((N, 384)), _fp((N, 20))),
        [_bs((NB, 384)), _bs((NB, 20))],
    )

    dec = jnp.transpose(atom[:, AC:4 * AC].reshape(N, 3, AC), (0, 2, 1))
    return dec, logits


def _gather(table, idx):
    return _sc_gather(table, idx)


# speedup vs baseline: 1.0220x; 1.0220x over previous
import functools

import jax
import jax.numpy as jnp
import numpy as np
from jax import lax
from jax.experimental import pallas as pl
from jax.experimental.pallas import tpu as pltpu
from jax.experimental.pallas import tpu_sc as plsc

N = 2048; K = 16; L = 9; H = 32; H2 = 64; NH = 8; NL = 4; AC = 91; EC = 32
E = N * K
NB = 256
EB = NB * K
GRID = N // NB
F32 = jnp.float32


def _fp(shape):
    return jax.ShapeDtypeStruct(shape, F32)


def _bs(shape, edge=False):
    nd = len(shape)
    return pl.BlockSpec(shape, lambda i: (i,) + (0,) * (nd - 1))


def _ws(shape):
    nd = len(shape)
    return pl.BlockSpec(shape, lambda i: (0,) * nd)


def _brd16(x):
    nb = x.shape[0]
    return jnp.broadcast_to(x[:, None, :], (nb, K, x.shape[1])).reshape(nb * K, x.shape[1])


def _sc_gather(table, idx2):
    T, D = table.shape
    nrow, gc = idx2.shape
    Etot = nrow * gc
    NW = 32
    nch = nrow // NW
    mesh = plsc.VectorSubcoreMesh(core_axis_name="c", subcore_axis_name="s")

    @functools.partial(
        pl.kernel, mesh=mesh,
        out_type=jax.ShapeDtypeStruct((Etot, D), F32),
        scratch_types=[
            pltpu.VMEM((nch, gc), jnp.int32),
            pltpu.VMEM((gc, D), F32),
            pltpu.VMEM((gc, D), F32),
            pltpu.SemaphoreType.DMA,
            pltpu.SemaphoreType.DMA,
        ],
    )
    def k(table_hbm, idx_hbm, out_hbm, idx_v, buf0, buf1, gsem, wsem):
        wid = lax.axis_index("s") * 2 + lax.axis_index("c")
        base = wid * nch
        pltpu.sync_copy(idx_hbm.at[pl.ds(base, nch)], idx_v)
        bufs = (buf0, buf1)
        g = [None, None]
        w = [None, None]
        for c in range(nch):
            b = c % 2
            if w[b] is not None:
                w[b].wait()
            g[b] = pltpu.async_copy(table_hbm.at[idx_v.at[c]], bufs[b], gsem)
            ob = 1 - b
            if c >= 1:
                g[ob].wait()
                w[ob] = pltpu.async_copy(
                    bufs[ob], out_hbm.at[pl.ds((base + c - 1) * gc, gc)], wsem)
        last = (nch - 1) % 2
        g[last].wait()
        w[last] = pltpu.async_copy(
            bufs[last], out_hbm.at[pl.ds((base + nch - 1) * gc, gc)], wsem)
        if w[1 - last] is not None:
            w[1 - last].wait()
        w[last].wait()

    return k(table, idx2)


def _pack_bf16(vlo, vhi):
    ulo = lax.bitcast_convert_type(vlo, jnp.uint32) >> 16
    uhi = lax.bitcast_convert_type(vhi, jnp.uint32) & jnp.uint32(0xFFFF0000)
    return lax.bitcast_convert_type(uhi | ulo, F32)


def _unpack_bf16(w):
    u = lax.bitcast_convert_type(w, jnp.uint32)
    vlo = lax.bitcast_convert_type(u << 16, F32)
    vhi = lax.bitcast_convert_type(u & jnp.uint32(0xFFFF0000), F32)
    return vlo, vhi


def _to_rows(x, w):
    return jnp.concatenate([x[:, w * l:w * (l + 1)] for l in range(L)], axis=0)


def _from_rows(q, w):
    nb = q.shape[0] // L
    return jnp.concatenate([q[nb * l:nb * (l + 1), :] for l in range(L)], axis=1)


def _geom_body(xT, bbT, mk, bbf_o, xm4_o, xmT_o):
    eps = 1e-8
    xr = xT[...]
    X = bbT[...]
    m = mk[...]
    laneI = lax.broadcasted_iota(jnp.int32, (1, N), 1)
    zero1 = jnp.zeros((1, 1), F32)

    def row(A, i):
        return A[i:i + 1, :]

    def shiftL(r):
        return jnp.concatenate([r[:, 1:], zero1], axis=1)

    def shiftR(r):
        return jnp.concatenate([zero1, r[:, :-1]], axis=1)

    xmrows = [jnp.where(m > 0, 1e9, row(xr, c)) for c in range(3)]
    xmT_o[...] = jnp.concatenate(xmrows + [jnp.zeros((5, N), F32)], axis=0)
    xm4_o[...] = jnp.transpose(
        jnp.concatenate(xmrows + [jnp.zeros((1, N), F32)], axis=0), (1, 0))

    dx = [shiftL(row(xr, c)) - row(xr, c) for c in range(3)]
    nrmf = jnp.sqrt(dx[0] * dx[0] + dx[1] * dx[1] + dx[2] * dx[2])
    fu = [d / (nrmf + eps) for d in dx]
    fwd = [jnp.where(laneI <= N - 2, f, 0.0) for f in fu]
    bwd = [-shiftR(f) for f in fwd]

    bv = [row(X, 3 + c) - row(X, 0 + c) for c in range(3)]
    cv = [row(X, 6 + c) - row(X, 3 + c) for c in range(3)]
    av = [bv[1] * cv[2] - bv[2] * cv[1],
          bv[2] * cv[0] - bv[0] * cv[2],
          bv[0] * cv[1] - bv[1] * cv[0]]
    vcb = [(-0.58273431 * av[c] + 0.56802827 * bv[c] - 0.54067466 * cv[c]
            + row(X, 3 + c)) - row(xr, c) for c in range(3)]

    def sh1(A):
        return jnp.concatenate([A[1:2], A[2:3], shiftL(A[0:1])], axis=0)

    Xf = [jnp.concatenate([row(X, c), row(X, 3 + c), row(X, 6 + c)], axis=0)
          for c in range(3)]
    dXf = [sh1(Xf[c]) - Xf[c] for c in range(3)]
    nrm = jnp.sqrt(dXf[0] * dXf[0] + dXf[1] * dXf[1] + dXf[2] * dXf[2])
    U = [d / (nrm + eps) for d in dXf]
    u1 = [sh1(U[c]) for c in range(3)]
    u0 = [sh1(u1[c]) for c in range(3)]

    def crossn(a, b):
        c0 = a[1] * b[2] - a[2] * b[1]
        c1 = a[2] * b[0] - a[0] * b[2]
        c2 = a[0] * b[1] - a[1] * b[0]
        nn = jnp.sqrt(c0 * c0 + c1 * c1 + c2 * c2)
        return [c0 / (nn + eps), c1 / (nn + eps), c2 / (nn + eps)]

    n2 = crossn(U, u1)
    n1 = crossn(u1, u0)
    cosD = jnp.clip(n2[0] * n1[0] + n2[1] * n1[1] + n2[2] * n1[2],
                    -1 + 1e-7, 1 - 1e-7)
    sgn = jnp.sign(U[0] * n1[0] + U[1] * n1[1] + U[2] * n1[2])
    sinD = sgn * jnp.sqrt(1.0 - cosD * cosD)
    valid = laneI <= N - 2
    cosV = jnp.where(valid, cosD, 1.0)
    sinV = jnp.where(valid, sinD, 0.0)
    cp0 = jnp.where(laneI == 0, 1.0, shiftR(cosV[2:3]))
    sp0 = shiftR(sinV[2:3])
    cosP = [cp0, cosV[0:1], cosV[1:2]]
    sinP = [sp0, sinV[0:1], sinV[1:2]]

    rows = cosP + sinP + [jnp.zeros((1, N), F32)]
    for c in range(3):
        l1 = [row(X, a * 3 + c) - row(xr, c) for a in range(4)]
        l1 += [fwd[c], bwd[c], vcb[c]]
        rows += [jnp.nan_to_num(r) for r in l1]
    rows += [jnp.zeros((4, N), F32)]
    bbf_o[...] = jnp.transpose(jnp.concatenate(rows, axis=0), (1, 0))


RK = 256


def _knn_body(xm4, xmT, bbf, W1, nbr_o, dist_o, rel_o, bbp_o):
    i = pl.program_id(0)
    xb = xm4[...]
    xT = xmT[...]
    d2 = ((xb[:, 0:1] - xT[0:1, :]) ** 2 + (xb[:, 1:2] - xT[1:2, :]) ** 2
          + (xb[:, 2:3] - xT[2:3, :]) ** 2)
    rowg = i * RK + lax.broadcasted_iota(jnp.int32, (RK, 1), 0)
    colI = lax.broadcasted_iota(jnp.int32, (RK, N), 1)
    d2 = d2 + jnp.where(colI == rowg, 1e12, 0.0)
    idxs = []
    ds = []
    for _ in range(K):
        m = jnp.min(d2, axis=1, keepdims=True)
        j = jnp.min(jnp.where(d2 == m, colI, N), axis=1, keepdims=True)
        idxs.append(j)
        ds.append(m)
        d2 = jnp.where(colI == j, jnp.float32(jnp.inf), d2)
    nbr = jnp.concatenate(idxs, axis=1)
    nbr_o[...] = nbr
    dist_o[...] = jnp.sqrt(jnp.concatenate(ds, axis=1))
    rel_o[...] = (nbr - rowg).astype(F32)
    bbp_o[...] = jnp.dot(bbf[...], W1[...], preferred_element_type=F32)


def _embed_body(g0, dist, rel, latf, W2, beb, Lv0, quad0,
                ef_o, res_o, t1_o, sdst_o):
    g0v = g0[...][:, :L * H]
    d = dist[...]
    r = rel[...]
    mu = lax.broadcasted_iota(jnp.int32, (1, 16), 1).astype(F32) * (20.0 / 15.0)
    sig = 20.0 / 16.0
    rbf = jnp.exp(-(((d - mu) / sig) ** 2))
    freq = jnp.exp(lax.broadcasted_iota(jnp.int32, (1, 8), 1).astype(F32) * 2.0 * (-np.log(10000.0) / 16.0))
    ang = r * freq
    kk = jnp.round(ang * 0.15915494309189535)
    t = (ang - kk * 6.2831855) - kk * (-1.7484556e-7)
    t2 = t * t
    sn = t * (1 + t2 * (-1.0 / 6 + t2 * (1.0 / 120 + t2 * (-1.0 / 5040
         + t2 * (1.0 / 362880 + t2 * (-1.0 / 39916800 + t2 * (1.0 / 6227020800)))))))
    cs = 1 + t2 * (-0.5 + t2 * (1.0 / 24 + t2 * (-1.0 / 720 + t2 * (1.0 / 40320
         + t2 * (-1.0 / 3628800 + t2 * (1.0 / 479001600 + t2 * (-1.0 / 87178291200)))))))
    ef = jnp.concatenate([rbf, cs, sn], axis=1)
    ef_o[...] = ef
    m = jax.nn.relu(g0v + jnp.dot(ef, W2[...], preferred_element_type=F32) + beb[...])
    bbe = jnp.mean(m.reshape(NB, K, L * H), axis=1)
    la = latf[...]
    pieces = []
    for l in range(L):
        pieces.append(bbe[:, l * H:(l + 1) * H])
        pieces.append(la[:, l * H:(l + 1) * H])
    res = jnp.concatenate(pieces, axis=1)
    res_o[...] = res
    v0 = _from_rows(jnp.dot(_to_rows(res, H2), Lv0[...], preferred_element_type=F32), H2)
    quad = jnp.dot(res[:, :H2], quad0[...], preferred_element_type=F32)
    vp = _pack_bf16(v0[:, :288], v0[:, 288:])
    t1_o[...] = jnp.concatenate(
        [quad[:, :32], jnp.zeros((NB, 32), F32), vp, jnp.zeros((NB, 32), F32)], axis=1)
    sdst_o[...] = quad[:, 32:64]


def _layer_body(first, last, g, ef_p, p_p, sdst, res, A1e, La2, Lo, Lf1, Lf2,
                LvN, quadW, E1e, P576,
                res_o, t1_o, sdst_o, ef_o, p_o):
    gb = g[...]
    vlo, vhi = _unpack_bf16(gb[:, 64:352])
    v576 = jnp.concatenate([vlo, vhi], axis=1)
    if first:
        ef = ef_p[...]
    else:
        ef = ef_p[...] + jnp.tanh(gb[:, 32:64] + p_p[...])
    ef_o[...] = ef
    a1 = jax.nn.relu(gb[:, 0:32] + _brd16(sdst[...])
                     + jnp.dot(ef, A1e[...], preferred_element_type=F32))
    a = jnp.dot(a1, La2[...], preferred_element_type=F32)
    a3 = a.reshape(NB, K, NH)
    amax = jnp.max(a3, axis=1, keepdims=True)
    ex = jnp.exp(a3 - amax)
    attn = (ex / jnp.sum(ex, axis=1, keepdims=True)).reshape(NB * K, NH)
    att576 = jnp.dot(attn, P576[...], preferred_element_type=F32)
    msg = jnp.sum((v576 * att576).reshape(NB, K, L * H2), axis=1)
    msgr = _to_rows(msg, H2)
    if last:
        resp = _from_rows(jnp.dot(msgr, Lo[...], preferred_element_type=F32), H)
        res_o[...] = jnp.dot(resp, LvN[...], preferred_element_type=F32)
        return
    resm = res[...] + _from_rows(jnp.dot(msgr, Lo[...], preferred_element_type=F32), H2)
    resmr = _to_rows(resm, H2)
    ffh = jax.nn.gelu(jnp.dot(resmr, Lf1[...], preferred_element_type=F32))
    resn = resm + _from_rows(jnp.dot(ffh, Lf2[...], preferred_element_type=F32), H2)
    res_o[...] = resn
    vn = _from_rows(jnp.dot(_to_rows(resn, H2), LvN[...], preferred_element_type=F32), H2)
    quad = jnp.dot(resn[:, :H2], quadW[...], preferred_element_type=F32)
    vp = _pack_bf16(vn[:, :288], vn[:, 288:])
    t1_o[...] = jnp.concatenate(
        [quad[:, 0:32], quad[:, 32:64], vp, jnp.zeros((NB, 32), F32)], axis=1)
    sdst_o[...] = quad[:, 64:96]
    p_o[...] = _brd16(quad[:, 96:128]) + jnp.dot(ef, E1e[...], preferred_element_type=F32)


def _out_body(g5, ef4, Wo2, boa, lng, lnb, Ws1, bs1, Ws2, bs2, atom_o, logits_o):
    m2 = jax.nn.relu(g5[...] + jnp.dot(ef4[...], Wo2[...], preferred_element_type=F32) + boa[...])
    atom = jnp.mean(m2.reshape(NB, K, 384), axis=1)
    atom_o[...] = atom
    invf = atom[:, :AC]
    mu = jnp.mean(invf, axis=-1, keepdims=True)
    xc = invf - mu
    var = jnp.mean(xc * xc, axis=-1, keepdims=True)
    h = xc / jnp.sqrt(var + 1e-5) * lng[...] + lnb[...]
    h = jax.nn.relu(jnp.dot(h, Ws1[...], preferred_element_type=F32) + bs1[...])
    lg = jnp.dot(h, Ws2[...], preferred_element_type=F32) + bs2[...]
    mx = jnp.max(lg, axis=-1, keepdims=True)
    s = lg - mx
    logits_o[...] = s - jnp.log(jnp.sum(jnp.exp(s), axis=-1, keepdims=True))


def _tc_call(body, in_arrs, in_specs, out_shapes, out_specs):
    return pl.pallas_call(
        body,
        grid=(GRID,),
        in_specs=in_specs,
        out_specs=out_specs,
        out_shape=out_shapes,
    )(*in_arrs)


def kernel(x, bb, x_mask, latent, W_eb, b_eb, La1, La2, Lv, Lo, Lf1, Lf2, Le,
           Pa1, Pa2, Pv, Po, W_oa, b_oa, ln_g, ln_b, Ws1, bs1, Ws2, bs2):
    xT8 = jnp.pad(x.T, ((0, 5), (0, 0)))
    bbT16 = jnp.pad(bb.transpose(1, 2, 0).reshape(12, N), ((0, 4), (0, 0)))
    mk = x_mask.astype(F32).reshape(1, N)
    bbf, xm4, xmT = pl.pallas_call(
        _geom_body,
        out_shape=(_fp((N, 32)), _fp((N, 4)), _fp((8, N))),
    )(xT8, bbT16, mk)


    W1 = jnp.pad(W_eb[:28], ((0, 4), (0, 96)))
    W2 = W_eb[28:]
    beb = b_eb.reshape(1, L * H)
    A1s = La1[:, :H2]; A1d = La1[:, H2:2 * H2]; A1e = La1[:, 2 * H2:]
    E1s = Le[:, :H2]; E1d = Le[:, H2:2 * H2]; E1e = Le[:, 2 * H2:]
    P1s = Pa1[:H2]; P1d = Pa1[H2:2 * H2]; P1e = Pa1[2 * H2:]
    Wo1 = jnp.pad(W_oa[:L * H], ((0, 0), (0, 20)))
    Wo2 = jnp.pad(W_oa[L * H:], ((0, 0), (0, 20)))
    boa = jnp.pad(b_oa, (0, 20)).reshape(1, 384)
    fcol = np.arange(L * H2)
    P576 = jnp.asarray((((fcol % H2) // 8)[None, :] == np.arange(NH)[:, None]).astype(np.float32))
    quad0 = jnp.concatenate([A1s[0], A1d[0]], axis=1)
    latf = latent.reshape(N, L * H)

    nbr, dist, rel, bbp = pl.pallas_call(
        _knn_body,
        grid=(N // RK,),
        in_specs=[_bs((RK, 4)), _ws((8, N)), _bs((RK, 32)), _ws((32, 384))],
        out_specs=[_bs((RK, K)), _bs((RK, K)), _bs((RK, K)), _bs((RK, 384))],
        out_shape=(jax.ShapeDtypeStruct((N, K), jnp.int32),
                   _fp((N, K)), _fp((N, K)), _fp((N, 384))),
    )(xm4, xmT, bbf, W1)
    idx2 = nbr.reshape(E // 64, 64)
    dist_e = dist.reshape(E, 1)
    rel_e = rel.reshape(E, 1)
    g0 = _gather(bbp, idx2)

    ef0, res0, t1, sdst = _tc_call(
        _embed_body,
        [g0, dist_e, rel_e, latf, W2, beb, Lv[0], quad0],
        [_bs((EB, 384)), _bs((EB, 1)), _bs((EB, 1)), _bs((NB, 288)),
         _ws((32, 288)), _ws((1, 288)), _ws((64, 64)), _ws((64, 64))],
        (_fp((E, 32)), _fp((N, 576)), _fp((N, 384)), _fp((N, 32))),
        [_bs((EB, 32)), _bs((NB, 576)), _bs((NB, 384)), _bs((NB, 32))],
    )

    res, ef, p = res0, ef0, None
    for i in range(NL + 1):
        first = i == 0
        last = i == NL
        g = _gather(t1, idx2)
        if last:
            lvn = Wo1
            quadw = jnp.zeros((64, 128), F32)
            e1e = jnp.zeros((32, 32), F32)
            lo_i, la2_i, a1e_i = Po, Pa2, P1e
            lf1_i = jnp.zeros((64, 32), F32); lf2_i = jnp.zeros((32, 64), F32)
        else:
            a1e_i, la2_i, lo_i, lf1_i, lf2_i = A1e[i], La2[i], Lo[i], Lf1[i], Lf2[i]
            if i == NL - 1:
                lvn = Pv
                quadw = jnp.concatenate([P1s, E1s[i], P1d, E1d[i]], axis=1)
            else:
                lvn = Lv[i + 1]
                quadw = jnp.concatenate([A1s[i + 1], E1s[i], A1d[i + 1], E1d[i]], axis=1)
            e1e = E1e[i]
        p_in = ef if first else p
        body = functools.partial(_layer_body, first, last)
        outs = _tc_call(
            body,
            [g, ef, p_in, sdst, res, a1e_i, la2_i, lo_i, lf1_i, lf2_i,
             lvn, quadw, e1e, P576],
            [_bs((EB, 384)), _bs((EB, 32)), _bs((EB, 32)), _bs((NB, 32)),
             _bs((NB, 576)), _ws((32, 32)), _ws((32, NH)), _ws(lo_i.shape),
             _ws((64, 32)), _ws((32, 64)), _ws(lvn.shape), _ws((64, 128)),
             _ws((32, 32)), _ws((NH, 576))],
            (_fp((N, 384)) if last else _fp((N, 576)),
             _fp((N, 384)), _fp((N, 32)), _fp((E, 32)), _fp((E, 32))),
            [_bs((NB, 384)) if last else _bs((NB, 576)),
             _bs((NB, 384)), _bs((NB, 32)), _bs((EB, 32)), _bs((EB, 32))],
        )
        if last:
            oat, _, _, ef4, _ = outs
        else:
            res, t1, sdst, ef, p = outs

    g5 = _gather(oat, idx2)
    atom, logits = _tc_call(
        _out_body,
        [g5, ef4, Wo2, boa, ln_g.reshape(1, AC), ln_b.reshape(1, AC),
         Ws1, bs1.reshape(1, AC), Ws2, bs2.reshape(1, 20)],
        [_bs((EB, 384)), _bs((EB, 32)), _ws((32, 384)), _ws((1, 384)),
         _ws((1, AC)), _ws((1, AC)), _ws((AC, AC)), _ws((1, AC)),
         _ws((AC, 20)), _ws((1, 20))],
        (_fp((N, 384)), _fp((N, 20))),
        [_bs((NB, 384)), _bs((NB, 20))],
    )

    dec = jnp.transpose(atom[:, AC:4 * AC].reshape(N, 3, AC), (0, 2, 1))
    return dec, logits


def _gather(table, idx):
    return _sc_gather(table, idx)
